# Initial kernel scaffold; baseline (speedup 1.0000x reference)
import jax
import jax.numpy as jnp
from jax.experimental import pallas as pl


def kernel(pred_logits, pred_boxes, target_sizes, select_id):
    raise NotImplementedError("write your pallas kernel here")


# speedup vs baseline: 18.0938x; 18.0938x over previous
import functools

import jax
import jax.numpy as jnp
from jax.experimental import pallas as pl


def _rowmax_body(x_ref, o_ref):
    o_ref[...] = jnp.max(x_ref[...], axis=2)[:, None, :]


@functools.partial(jax.jit, static_argnums=())
def _rowmax(pred_logits):
    B, N, C = pred_logits.shape
    NBLK = 2000
    NB = N // NBLK
    out = pl.pallas_call(
        _rowmax_body,
        grid=(B, NB),
        in_specs=[pl.BlockSpec((1, NBLK, C), lambda b, i: (b, i, 0))],
        out_specs=pl.BlockSpec((1, 1, NBLK), lambda b, i: (b * NB + i, 0, 0)),
        out_shape=jax.ShapeDtypeStruct((B * NB, 1, NBLK), jnp.float32),
    )(pred_logits)
    return out.reshape(B, N)


def kernel(pred_logits, pred_boxes, target_sizes, select_id):
    B, N, C = pred_logits.shape
    TOPK = 300
    K1 = 384

    rowmax = _rowmax(pred_logits)

    _, cand_pos = jax.lax.top_k(rowmax, K1)
    cand_pos_sorted = jnp.sort(cand_pos, axis=1)
    cand = jnp.take_along_axis(
        pred_logits, cand_pos_sorted[:, :, None], axis=1)
    cand_flat = cand.reshape(B, K1 * C)
    vals, loc = jax.lax.top_k(cand_flat, TOPK)
    n_idx = jnp.take_along_axis(cand_pos_sorted, loc // C, axis=1)
    topk_indexes = n_idx * C + (loc % C)

    scores = jax.nn.sigmoid(vals)
    labels = jnp.where(n_idx < 300, jnp.float32(select_id), 0.0)

    cx = jnp.take_along_axis(pred_boxes[..., 0], n_idx, axis=1)
    cy = jnp.take_along_axis(pred_boxes[..., 1], n_idx, axis=1)
    w = jnp.take_along_axis(pred_boxes[..., 2], n_idx, axis=1)
    h = jnp.take_along_axis(pred_boxes[..., 3], n_idx, axis=1)
    img_h = target_sizes[:, 0].astype(jnp.float32)[:, None]
    img_w = target_sizes[:, 1].astype(jnp.float32)[:, None]
    boxes = jnp.stack([
        (cx - 0.5 * w) * img_w,
        (cy - 0.5 * h) * img_h,
        (cx + 0.5 * w) * img_w,
        (cy + 0.5 * h) * img_h,
    ], axis=-1)
    return (scores, labels, boxes, topk_indexes)
